# baseline (speedup 1.0000x reference)
import functools

import jax
import jax.numpy as jnp
from jax import lax
from jax.experimental import pallas as pl
from jax.experimental.pallas import tpu as pltpu
from jax.experimental.pallas import tpu_sc as plsc

N = 10000
D = 256
DE = 16
L = 3

NC = 2
NS = 16
HALF = D // 2
CHUNK = 128
CHUNKS_B = 80
CHUNKS_A = 40
E_PAD = NS * CHUNKS_B * CHUNK
N_PAD = 10240
RPT = N_PAD // NS
EF_W = 128

_MESH = plsc.VectorSubcoreMesh(core_axis_name="c", subcore_axis_name="s")


@functools.partial(
    pl.kernel,
    out_type=jax.ShapeDtypeStruct((NC, N_PAD, EF_W), jnp.float32),
    mesh=_MESH,
    scratch_types=[
        pltpu.VMEM((CHUNKS_A, CHUNK), jnp.int32),
        pltpu.VMEM((CHUNK, EF_W), jnp.float32),
        pltpu.VMEM((CHUNK, EF_W), jnp.float32),
        pltpu.VMEM_SHARED((N_PAD, EF_W), jnp.float32),
        pltpu.SemaphoreType.DMA,
        pltpu.SemaphoreType.DMA,
        pltpu.SemaphoreType.DMA,
        pltpu.SemaphoreType.DMA,
    ],
)
def _edge_segsum(ef_hbm, dst_hbm, zero_hbm, out_hbm, dst_v, ef0, ef1, acc,
                 gs0, gs1, ss0, ss1):
    c = lax.axis_index("c")
    s = lax.axis_index("s")
    pltpu.sync_copy(zero_hbm, acc.at[pl.ds(s * RPT, RPT)])
    pltpu.sync_copy(dst_hbm.at[c, s], dst_v)
    plsc.subcore_barrier()

    bufs = (ef0, ef1)
    gsems = (gs0, gs1)
    ssems = (ss0, ss1)
    pltpu.async_copy(ef_hbm.at[c, s, 0], ef0, gs0)
    pltpu.async_copy(ef_hbm.at[c, s, 1], ef1, gs1)

    def body(i, carry):
        j = i * 2
        for b in range(2):
            jj = j + b
            pltpu.make_async_copy(ef_hbm.at[c, s, jj], bufs[b], gsems[b]).wait()
            pltpu.async_copy(bufs[b], acc.at[dst_v.at[jj]], ssems[b], add=True)
        for b in range(2):
            jj = j + b
            pltpu.make_async_copy(bufs[b], acc.at[dst_v.at[jj]], ssems[b]).wait()

            @pl.when(jj + 2 < CHUNKS_A)
            def _():
                pltpu.async_copy(ef_hbm.at[c, s, jj + 2], bufs[b], gsems[b])

        return carry

    lax.fori_loop(0, CHUNKS_A // 2, body, 0)
    plsc.subcore_barrier()
    pltpu.sync_copy(acc.at[pl.ds(s * RPT, RPT)], out_hbm.at[c, pl.ds(s * RPT, RPT)])


@functools.partial(
    pl.kernel,
    out_type=(
        jax.ShapeDtypeStruct((N_PAD, HALF), jnp.float32),
        jax.ShapeDtypeStruct((N_PAD, HALF), jnp.float32),
    ),
    mesh=_MESH,
    scratch_types=[
        pltpu.VMEM((CHUNKS_B // 2, CHUNK), jnp.int32),
        pltpu.VMEM((CHUNKS_B // 2, CHUNK), jnp.int32),
        pltpu.VMEM((CHUNK, HALF), jnp.float32),
        pltpu.VMEM((CHUNK, HALF), jnp.float32),
        pltpu.VMEM_SHARED((N_PAD, HALF), jnp.float32),
        pltpu.SemaphoreType.DMA,
        pltpu.SemaphoreType.DMA,
    ],
)
def _gather_segsum(xlo_hbm, xhi_hbm, src_hbm, dst_hbm, zero_hbm,
                   glo_hbm, ghi_hbm,
                   src_v, dst_v, rows0, rows1, acc, gs0, gs1):
    c = lax.axis_index("c")
    s = lax.axis_index("s")
    half_chunks = CHUNKS_B // 2
    pltpu.sync_copy(zero_hbm, acc.at[pl.ds(s * RPT, RPT)])
    plsc.subcore_barrier()

    def run(x_hbm, g_hbm):
        bufs = (rows0, rows1)
        gsems = (gs0, gs1)
        nb = 2
        for phase in range(2):
            base = phase * half_chunks
            pltpu.sync_copy(src_hbm.at[s, pl.ds(base, half_chunks)], src_v)
            pltpu.sync_copy(dst_hbm.at[s, pl.ds(base, half_chunks)], dst_v)
            for b in range(nb):
                pltpu.async_copy(x_hbm.at[src_v.at[b]], bufs[b], gsems[b])

            def outer(i, carry):
                j = i * nb
                for b in range(nb):
                    jj = j + b
                    pltpu.make_async_copy(x_hbm.at[src_v.at[jj]], bufs[b], gsems[b]).wait()
                    pltpu.sync_copy(bufs[b], acc.at[dst_v.at[jj]], add=True)

                    @pl.when(jj + nb < half_chunks)
                    def _():
                        pltpu.async_copy(x_hbm.at[src_v.at[jj + nb]], bufs[b], gsems[b])

                return carry

            lax.fori_loop(0, half_chunks // nb, outer, 0)
        plsc.subcore_barrier()
        pltpu.sync_copy(acc.at[pl.ds(s * RPT, RPT)], g_hbm.at[pl.ds(s * RPT, RPT)])

    @pl.when(c == 0)
    def _():
        run(xlo_hbm, glo_hbm)

    @pl.when(c == 1)
    def _():
        run(xhi_hbm, ghi_hbm)


BLK = 2000


def _tc_body(split, xlo, xhi, glo, ghi, ge, wx0, wx1, wea, gam, bet, *outs):
    hp = lax.Precision.HIGHEST
    f32 = jnp.float32
    agg = (
        jnp.dot(glo[...], wx0[...], precision=hp, preferred_element_type=f32)
        + jnp.dot(ghi[...], wx1[...], precision=hp, preferred_element_type=f32)
        + jnp.dot(ge[0] + ge[1], wea[...], precision=hp, preferred_element_type=f32)
    )
    x = jnp.concatenate([xlo[...], xhi[...]], axis=1)
    h = x + agg
    mu = jnp.mean(h, axis=1, keepdims=True)
    hc = h - mu
    var = jnp.mean(hc * hc, axis=1, keepdims=True)
    y = hc * lax.rsqrt(var + 1e-5) * gam[...] + bet[...]
    y = jnp.maximum(y, 0.0)
    if split:
        outs[0][...] = y[:, :HALF]
        outs[1][...] = y[:, HALF:]
    else:
        outs[0][...] = y


def _make_tc(split):
    in_specs = [
        pl.BlockSpec((BLK, HALF), lambda i: (i, 0)),
        pl.BlockSpec((BLK, HALF), lambda i: (i, 0)),
        pl.BlockSpec((BLK, HALF), lambda i: (i, 0)),
        pl.BlockSpec((BLK, HALF), lambda i: (i, 0)),
        pl.BlockSpec((NC, BLK, EF_W), lambda i: (0, i, 0)),
        pl.BlockSpec((HALF, D), lambda i: (0, 0)),
        pl.BlockSpec((HALF, D), lambda i: (0, 0)),
        pl.BlockSpec((EF_W, D), lambda i: (0, 0)),
        pl.BlockSpec((1, D), lambda i: (0, 0)),
        pl.BlockSpec((1, D), lambda i: (0, 0)),
    ]
    if split:
        out_shape = (
            jax.ShapeDtypeStruct((N, HALF), jnp.float32),
            jax.ShapeDtypeStruct((N, HALF), jnp.float32),
        )
        out_specs = (
            pl.BlockSpec((BLK, HALF), lambda i: (i, 0)),
            pl.BlockSpec((BLK, HALF), lambda i: (i, 0)),
        )
    else:
        out_shape = jax.ShapeDtypeStruct((N, D), jnp.float32)
        out_specs = pl.BlockSpec((BLK, D), lambda i: (i, 0))
    return pl.pallas_call(
        functools.partial(_tc_body, split),
        grid=(N // BLK,),
        in_specs=in_specs,
        out_specs=out_specs,
        out_shape=out_shape,
    )


_tc_split = _make_tc(True)
_tc_final = _make_tc(False)


def kernel(node_features, edge_index, edge_features, Ws, bs, gammas, betas):
    src = edge_index[0].astype(jnp.int32)
    dst = edge_index[1].astype(jnp.int32)
    e = src.shape[0]
    pad = E_PAD - e

    srcp = jnp.concatenate([src, jnp.zeros((pad,), jnp.int32)])
    dstp = jnp.concatenate([dst, jnp.full((pad,), N, jnp.int32)])
    order = jnp.argsort(srcp)
    src_s = srcp[order]
    dst_s = dstp[order]
    src_b = src_s.reshape(NS, CHUNKS_B, CHUNK)
    dst_b = dst_s.reshape(NS, CHUNKS_B, CHUNK)
    dst_a = dstp.reshape(NC, NS, CHUNKS_A, CHUNK)

    ef_aug = jnp.concatenate(
        [edge_features,
         jnp.ones((e, 1), jnp.float32),
         jnp.zeros((e, EF_W - DE - 1), jnp.float32)], axis=1)
    ef_a = jnp.concatenate([ef_aug, jnp.zeros((pad, EF_W), jnp.float32)], axis=0)
    ef_a = ef_a.reshape(NC, NS, CHUNKS_A, CHUNK, EF_W)

    z_ef = jnp.zeros((RPT, EF_W), jnp.float32)
    z_half = jnp.zeros((RPT, HALF), jnp.float32)

    ge = _edge_segsum(ef_a, dst_a, z_ef)

    xlo = node_features[:, :HALF]
    xhi = node_features[:, HALF:]
    out = None
    for i in range(L):
        w = Ws[i]
        wx0 = w[:HALF]
        wx1 = w[HALF:D]
        wea = jnp.concatenate(
            [w[D:], bs[i][None, :], jnp.zeros((EF_W - DE - 1, D), jnp.float32)],
            axis=0)
        gam = gammas[i][None, :]
        bet = betas[i][None, :]
        glo, ghi = _gather_segsum(xlo, xhi, src_b, dst_b, z_half)
        if i < L - 1:
            xlo, xhi = _tc_split(xlo, xhi, glo, ghi, ge, wx0, wx1, wea, gam, bet)
        else:
            out = _tc_final(xlo, xhi, glo, ghi, ge, wx0, wx1, wea, gam, bet)
    return out


# speedup vs baseline: 1.4465x; 1.4465x over previous
import functools

import jax
import jax.numpy as jnp
from jax import lax
from jax.experimental import pallas as pl
from jax.experimental.pallas import tpu as pltpu
from jax.experimental.pallas import tpu_sc as plsc

N = 10000
D = 256
DE = 16
L = 3

NC = 2
NS = 16
HALF = D // 2
CHUNK = 128
CHUNKS_B = 80
CHUNKS_A = 40
E_PAD = NS * CHUNKS_B * CHUNK
N_PAD = 10240
RPT = N_PAD // NS
EF_W = 128

_MESH = plsc.VectorSubcoreMesh(core_axis_name="c", subcore_axis_name="s")


@functools.partial(
    pl.kernel,
    out_type=jax.ShapeDtypeStruct((NC, N_PAD, EF_W), jnp.float32),
    mesh=_MESH,
    scratch_types=[
        pltpu.VMEM((CHUNKS_A, CHUNK), jnp.int32),
        pltpu.VMEM((CHUNK, EF_W), jnp.float32),
        pltpu.VMEM((CHUNK, EF_W), jnp.float32),
        pltpu.VMEM_SHARED((N_PAD, EF_W), jnp.float32),
        pltpu.SemaphoreType.DMA,
        pltpu.SemaphoreType.DMA,
        pltpu.SemaphoreType.DMA,
        pltpu.SemaphoreType.DMA,
    ],
)
def _edge_segsum(ef_hbm, dst_hbm, zero_hbm, out_hbm, dst_v, ef0, ef1, acc,
                 gs0, gs1, ss0, ss1):
    c = lax.axis_index("c")
    s = lax.axis_index("s")
    pltpu.sync_copy(zero_hbm, acc.at[pl.ds(s * RPT, RPT)])
    pltpu.sync_copy(dst_hbm.at[c, s], dst_v)
    plsc.subcore_barrier()

    bufs = (ef0, ef1)
    gsems = (gs0, gs1)
    ssems = (ss0, ss1)
    pltpu.async_copy(ef_hbm.at[c, s, 0], ef0, gs0)
    pltpu.async_copy(ef_hbm.at[c, s, 1], ef1, gs1)

    def body(i, carry):
        j = i * 2
        for b in range(2):
            jj = j + b
            pltpu.make_async_copy(ef_hbm.at[c, s, jj], bufs[b], gsems[b]).wait()
            pltpu.async_copy(bufs[b], acc.at[dst_v.at[jj]], ssems[b], add=True)
        for b in range(2):
            jj = j + b
            pltpu.make_async_copy(bufs[b], acc.at[dst_v.at[jj]], ssems[b]).wait()

            @pl.when(jj + 2 < CHUNKS_A)
            def _():
                pltpu.async_copy(ef_hbm.at[c, s, jj + 2], bufs[b], gsems[b])

        return carry

    lax.fori_loop(0, CHUNKS_A // 2, body, 0)
    plsc.subcore_barrier()
    pltpu.sync_copy(acc.at[pl.ds(s * RPT, RPT)], out_hbm.at[c, pl.ds(s * RPT, RPT)])


@functools.partial(
    pl.kernel,
    out_type=(
        jax.ShapeDtypeStruct((N_PAD, HALF), jnp.float32),
        jax.ShapeDtypeStruct((N_PAD, HALF), jnp.float32),
    ),
    mesh=_MESH,
    scratch_types=[
        pltpu.VMEM((CHUNKS_B // 2, CHUNK), jnp.int32),
        pltpu.VMEM((CHUNKS_B // 2, CHUNK), jnp.int32),
        pltpu.VMEM((CHUNK, HALF), jnp.float32),
        pltpu.VMEM((CHUNK, HALF), jnp.float32),
        pltpu.VMEM_SHARED((N_PAD, HALF), jnp.float32),
        pltpu.SemaphoreType.DMA,
        pltpu.SemaphoreType.DMA,
    ],
)
def _gather_segsum(xlo_hbm, xhi_hbm, src_hbm, dst_hbm, zero_hbm,
                   glo_hbm, ghi_hbm,
                   src_v, dst_v, rows0, rows1, acc, gs0, gs1):
    c = lax.axis_index("c")
    s = lax.axis_index("s")
    half_chunks = CHUNKS_B // 2
    pltpu.sync_copy(zero_hbm, acc.at[pl.ds(s * RPT, RPT)])
    plsc.subcore_barrier()

    def run(x_hbm, g_hbm):
        bufs = (rows0, rows1)
        gsems = (gs0, gs1)
        nb = 2
        for phase in range(2):
            base = phase * half_chunks
            pltpu.sync_copy(src_hbm.at[s, pl.ds(base, half_chunks)], src_v)
            pltpu.sync_copy(dst_hbm.at[s, pl.ds(base, half_chunks)], dst_v)
            for b in range(nb):
                pltpu.async_copy(x_hbm.at[src_v.at[b]], bufs[b], gsems[b])

            def outer(i, carry):
                j = i * nb
                for b in range(nb):
                    jj = j + b
                    pltpu.make_async_copy(x_hbm.at[src_v.at[jj]], bufs[b], gsems[b]).wait()
                    pltpu.sync_copy(bufs[b], acc.at[dst_v.at[jj]], add=True)

                    @pl.when(jj + nb < half_chunks)
                    def _():
                        pltpu.async_copy(x_hbm.at[src_v.at[jj + nb]], bufs[b], gsems[b])

                return carry

            lax.fori_loop(0, half_chunks // nb, outer, 0)
        plsc.subcore_barrier()
        pltpu.sync_copy(acc.at[pl.ds(s * RPT, RPT)], g_hbm.at[pl.ds(s * RPT, RPT)])

    @pl.when(c == 0)
    def _():
        run(xlo_hbm, glo_hbm)

    @pl.when(c == 1)
    def _():
        run(xhi_hbm, ghi_hbm)


BLK = 2000


def _tc_body(split, xlo, xhi, glo, ghi, ge, wx0, wx1, wea, gam, bet, *outs):
    hp = lax.Precision.HIGHEST
    f32 = jnp.float32
    agg = (
        jnp.dot(glo[...], wx0[...], precision=hp, preferred_element_type=f32)
        + jnp.dot(ghi[...], wx1[...], precision=hp, preferred_element_type=f32)
        + jnp.dot(ge[0] + ge[1], wea[...], precision=hp, preferred_element_type=f32)
    )
    x = jnp.concatenate([xlo[...], xhi[...]], axis=1)
    h = x + agg
    mu = jnp.mean(h, axis=1, keepdims=True)
    hc = h - mu
    var = jnp.mean(hc * hc, axis=1, keepdims=True)
    y = hc * lax.rsqrt(var + 1e-5) * gam[...] + bet[...]
    y = jnp.maximum(y, 0.0)
    if split:
        outs[0][...] = y[:, :HALF]
        outs[1][...] = y[:, HALF:]
    else:
        outs[0][...] = y


def _make_tc(split):
    in_specs = [
        pl.BlockSpec((BLK, HALF), lambda i: (i, 0)),
        pl.BlockSpec((BLK, HALF), lambda i: (i, 0)),
        pl.BlockSpec((BLK, HALF), lambda i: (i, 0)),
        pl.BlockSpec((BLK, HALF), lambda i: (i, 0)),
        pl.BlockSpec((NC, BLK, EF_W), lambda i: (0, i, 0)),
        pl.BlockSpec((HALF, D), lambda i: (0, 0)),
        pl.BlockSpec((HALF, D), lambda i: (0, 0)),
        pl.BlockSpec((EF_W, D), lambda i: (0, 0)),
        pl.BlockSpec((1, D), lambda i: (0, 0)),
        pl.BlockSpec((1, D), lambda i: (0, 0)),
    ]
    if split:
        out_shape = (
            jax.ShapeDtypeStruct((N, HALF), jnp.float32),
            jax.ShapeDtypeStruct((N, HALF), jnp.float32),
        )
        out_specs = (
            pl.BlockSpec((BLK, HALF), lambda i: (i, 0)),
            pl.BlockSpec((BLK, HALF), lambda i: (i, 0)),
        )
    else:
        out_shape = jax.ShapeDtypeStruct((N, D), jnp.float32)
        out_specs = pl.BlockSpec((BLK, D), lambda i: (i, 0))
    return pl.pallas_call(
        functools.partial(_tc_body, split),
        grid=(N // BLK,),
        in_specs=in_specs,
        out_specs=out_specs,
        out_shape=out_shape,
    )


_tc_split = _make_tc(True)
_tc_final = _make_tc(False)


def kernel(node_features, edge_index, edge_features, Ws, bs, gammas, betas):
    src = edge_index[0].astype(jnp.int32)
    dst = edge_index[1].astype(jnp.int32)
    e = src.shape[0]
    pad = E_PAD - e

    srcp = jnp.concatenate([src, jnp.zeros((pad,), jnp.int32)])
    dstp = jnp.concatenate([dst, jnp.full((pad,), N, jnp.int32)])
    src_b = srcp.reshape(NS, CHUNKS_B, CHUNK)
    dst_b = dstp.reshape(NS, CHUNKS_B, CHUNK)
    dst_a = dstp.reshape(NC, NS, CHUNKS_A, CHUNK)

    ef_aug = jnp.concatenate(
        [edge_features,
         jnp.ones((e, 1), jnp.float32),
         jnp.zeros((e, EF_W - DE - 1), jnp.float32)], axis=1)
    ef_a = jnp.concatenate([ef_aug, jnp.zeros((pad, EF_W), jnp.float32)], axis=0)
    ef_a = ef_a.reshape(NC, NS, CHUNKS_A, CHUNK, EF_W)

    z_ef = jnp.zeros((RPT, EF_W), jnp.float32)
    z_half = jnp.zeros((RPT, HALF), jnp.float32)

    ge = _edge_segsum(ef_a, dst_a, z_ef)

    xlo = node_features[:, :HALF]
    xhi = node_features[:, HALF:]
    out = None
    for i in range(L):
        w = Ws[i]
        wx0 = w[:HALF]
        wx1 = w[HALF:D]
        wea = jnp.concatenate(
            [w[D:], bs[i][None, :], jnp.zeros((EF_W - DE - 1, D), jnp.float32)],
            axis=0)
        gam = gammas[i][None, :]
        bet = betas[i][None, :]
        glo, ghi = _gather_segsum(xlo, xhi, src_b, dst_b, z_half)
        if i < L - 1:
            xlo, xhi = _tc_split(xlo, xhi, glo, ghi, ge, wx0, wx1, wea, gam, bet)
        else:
            out = _tc_final(xlo, xhi, glo, ghi, ge, wx0, wx1, wea, gam, bet)
    return out
